# Initial kernel scaffold; baseline (speedup 1.0000x reference)
import jax
import jax.numpy as jnp
from jax.experimental import pallas as pl


def kernel(x, edge_index, W1_l, b1_l, W1_r, W2_l, b2_l, W2_r):
    raise NotImplementedError("write your pallas kernel here")


# speedup vs baseline: 4.8804x; 4.8804x over previous
import functools

import jax
import jax.numpy as jnp
from jax import lax
from jax.experimental import pallas as pl
from jax.experimental.pallas import tpu as pltpu
from jax.experimental.pallas import tpu_sc as plsc

N_NODES = 10000
E = 320000
IN_DIM = 128
HID = 64

NC = 2
NS = 16
NW = NC * NS
SUB = 128
K = 4
CHUNK = SUB * K
CHUNKS_PER_W = 20
E_PAD = NW * CHUNK * CHUNKS_PER_W
IDXROWS_PER_W = E_PAD // SUB // NW
N_PAD = 10240
ROWS_PER_TILE = N_PAD // NS
DUMMY = N_NODES
RBLK = 512


def _tc_pre_body(x_ref, wl_ref, wr_ref, b_ref, p_ref, q_ref):
    xb = x_ref[...]
    p_ref[...] = jnp.dot(xb, wl_ref[...], preferred_element_type=jnp.float32)
    q_ref[...] = (jnp.dot(xb, wr_ref[...], preferred_element_type=jnp.float32)
                  + b_ref[...])


def _tc_pre(x_pad, W_l, W_r, b):
    grid = (N_PAD // RBLK,)
    return pl.pallas_call(
        _tc_pre_body,
        grid=grid,
        in_specs=[
            pl.BlockSpec((RBLK, IN_DIM), lambda i: (i, 0)),
            pl.BlockSpec((IN_DIM, HID), lambda i: (0, 0)),
            pl.BlockSpec((IN_DIM, HID), lambda i: (0, 0)),
            pl.BlockSpec((1, HID), lambda i: (0, 0)),
        ],
        out_specs=[
            pl.BlockSpec((RBLK, HID), lambda i: (i, 0)),
            pl.BlockSpec((RBLK, HID), lambda i: (i, 0)),
        ],
        out_shape=[
            jax.ShapeDtypeStruct((N_PAD, HID), jnp.float32),
            jax.ShapeDtypeStruct((N_PAD, HID), jnp.float32),
        ],
    )(x_pad, W_l, W_r, b)


def _tc_mid_body(part_ref, degp_ref, q1_ref, wl_ref, wr_ref, b_ref,
                 p2_ref, q2_ref):
    agg = part_ref[0] + part_ref[1]
    deg = degp_ref[0, :, 0] + degp_ref[1, :, 0]
    mean = agg / jnp.maximum(deg, 1.0)[:, None]
    h = jnp.maximum(mean + q1_ref[...], 0.0)
    p2_ref[...] = jnp.dot(h, wl_ref[...], preferred_element_type=jnp.float32)
    q2_ref[...] = (jnp.dot(h, wr_ref[...], preferred_element_type=jnp.float32)
                   + b_ref[...])


def _tc_mid(part, degp, q1, W_l, W_r, b):
    grid = (N_PAD // RBLK,)
    return pl.pallas_call(
        _tc_mid_body,
        grid=grid,
        in_specs=[
            pl.BlockSpec((NC, RBLK, HID), lambda i: (0, i, 0)),
            pl.BlockSpec((NC, RBLK, 16), lambda i: (0, i, 0)),
            pl.BlockSpec((RBLK, HID), lambda i: (i, 0)),
            pl.BlockSpec((HID, HID), lambda i: (0, 0)),
            pl.BlockSpec((HID, HID), lambda i: (0, 0)),
            pl.BlockSpec((1, HID), lambda i: (0, 0)),
        ],
        out_specs=[
            pl.BlockSpec((RBLK, HID), lambda i: (i, 0)),
            pl.BlockSpec((RBLK, HID), lambda i: (i, 0)),
        ],
        out_shape=[
            jax.ShapeDtypeStruct((N_PAD, HID), jnp.float32),
            jax.ShapeDtypeStruct((N_PAD, HID), jnp.float32),
        ],
    )(part, degp, q1, W_l, W_r, b)


def _tc_out_body(part_ref, degp_ref, q2_ref, z_ref):
    agg = part_ref[0] + part_ref[1]
    deg = degp_ref[0, :, 0] + degp_ref[1, :, 0]
    z_ref[...] = agg / jnp.maximum(deg, 1.0)[:, None] + q2_ref[...]


def _tc_out(part, degp, q2):
    grid = (N_PAD // RBLK,)
    return pl.pallas_call(
        _tc_out_body,
        grid=grid,
        in_specs=[
            pl.BlockSpec((NC, RBLK, HID), lambda i: (0, i, 0)),
            pl.BlockSpec((NC, RBLK, 16), lambda i: (0, i, 0)),
            pl.BlockSpec((RBLK, HID), lambda i: (i, 0)),
        ],
        out_specs=pl.BlockSpec((RBLK, HID), lambda i: (i, 0)),
        out_shape=jax.ShapeDtypeStruct((N_PAD, HID), jnp.float32),
    )(part, degp, q2)


def _sc_agg_deg(table, src2d, dst2d, z64, z16, ones_h):
    mesh = plsc.VectorSubcoreMesh(core_axis_name="c", subcore_axis_name="s")

    @functools.partial(
        pl.kernel,
        mesh=mesh,
        out_type=[
            jax.ShapeDtypeStruct((NC, N_PAD, HID), jnp.float32),
            jax.ShapeDtypeStruct((NC, N_PAD, 16), jnp.float32),
        ],
        compiler_params=pltpu.CompilerParams(use_tc_tiling_on_sc=False),
        scratch_types=[
            pltpu.VMEM((K, SUB), jnp.int32),
            pltpu.VMEM((K, SUB), jnp.int32),
            pltpu.VMEM((CHUNK, HID), jnp.float32),
            pltpu.VMEM((SUB, 16), jnp.float32),
            pltpu.VMEM_SHARED((N_PAD, HID), jnp.float32),
            pltpu.VMEM_SHARED((N_PAD, 16), jnp.float32),
            pltpu.SemaphoreType.DMA,
        ],
    )
    def k(table_h, src_h, dst_h, z64_h, z16_h, ones_hh, part_out, deg_out,
          idx_s, idx_d, rows_v, ones_v, acc, dacc, sem):
        c = lax.axis_index("c")
        s = lax.axis_index("s")
        w = s * NC + c
        tile_rows = pl.ds(s * ROWS_PER_TILE, ROWS_PER_TILE)
        pltpu.sync_copy(z64_h, acc.at[tile_rows])
        pltpu.sync_copy(z16_h, dacc.at[tile_rows])
        pltpu.sync_copy(ones_hh, ones_v)
        plsc.subcore_barrier()

        def chunk_body(t, carry):
            row0 = w * IDXROWS_PER_W + t * K
            pltpu.sync_copy(src_h.at[pl.ds(row0, K)], idx_s)
            pltpu.sync_copy(dst_h.at[pl.ds(row0, K)], idx_d)
            descs = [
                pltpu.async_copy(table_h.at[idx_s.at[j]],
                                 rows_v.at[pl.ds(j * SUB, SUB)], sem)
                for j in range(K)
            ]
            for d in descs:
                d.wait()
            for j in range(K):
                pltpu.sync_copy(rows_v.at[pl.ds(j * SUB, SUB)],
                                acc.at[idx_d.at[j]], add=True)
                pltpu.sync_copy(ones_v, dacc.at[idx_d.at[j]], add=True)
            return carry

        lax.fori_loop(0, CHUNKS_PER_W, chunk_body, 0)
        plsc.subcore_barrier()
        pltpu.sync_copy(acc.at[tile_rows], part_out.at[c].at[tile_rows])
        pltpu.sync_copy(dacc.at[tile_rows], deg_out.at[c].at[tile_rows])

    return k(table, src2d, dst2d, z64, z16, ones_h)


def _sc_agg(table, src2d, dst2d, z64):
    mesh = plsc.VectorSubcoreMesh(core_axis_name="c", subcore_axis_name="s")

    @functools.partial(
        pl.kernel,
        mesh=mesh,
        out_type=jax.ShapeDtypeStruct((NC, N_PAD, HID), jnp.float32),
        compiler_params=pltpu.CompilerParams(use_tc_tiling_on_sc=False),
        scratch_types=[
            pltpu.VMEM((K, SUB), jnp.int32),
            pltpu.VMEM((K, SUB), jnp.int32),
            pltpu.VMEM((CHUNK, HID), jnp.float32),
            pltpu.VMEM_SHARED((N_PAD, HID), jnp.float32),
            pltpu.SemaphoreType.DMA,
        ],
    )
    def k(table_h, src_h, dst_h, z64_h, part_out,
          idx_s, idx_d, rows_v, acc, sem):
        c = lax.axis_index("c")
        s = lax.axis_index("s")
        w = s * NC + c
        tile_rows = pl.ds(s * ROWS_PER_TILE, ROWS_PER_TILE)
        pltpu.sync_copy(z64_h, acc.at[tile_rows])
        plsc.subcore_barrier()

        def chunk_body(t, carry):
            row0 = w * IDXROWS_PER_W + t * K
            pltpu.sync_copy(src_h.at[pl.ds(row0, K)], idx_s)
            pltpu.sync_copy(dst_h.at[pl.ds(row0, K)], idx_d)
            descs = [
                pltpu.async_copy(table_h.at[idx_s.at[j]],
                                 rows_v.at[pl.ds(j * SUB, SUB)], sem)
                for j in range(K)
            ]
            for d in descs:
                d.wait()
            for j in range(K):
                pltpu.sync_copy(rows_v.at[pl.ds(j * SUB, SUB)],
                                acc.at[idx_d.at[j]], add=True)
            return carry

        lax.fori_loop(0, CHUNKS_PER_W, chunk_body, 0)
        plsc.subcore_barrier()
        pltpu.sync_copy(acc.at[tile_rows], part_out.at[c].at[tile_rows])

    return k(table, src2d, dst2d, z64)


def kernel(x, edge_index, W1_l, b1_l, W1_r, W2_l, b2_l, W2_r):
    src = edge_index[0].astype(jnp.int32)
    dst = edge_index[1].astype(jnp.int32)
    pad = E_PAD - E
    fill = jnp.full((pad,), DUMMY, jnp.int32)
    src2d = jnp.concatenate([src, fill]).reshape(E_PAD // SUB, SUB)
    dst2d = jnp.concatenate([dst, fill]).reshape(E_PAD // SUB, SUB)
    x_pad = jnp.pad(x, ((0, N_PAD - N_NODES), (0, 0)))
    z64 = jnp.zeros((ROWS_PER_TILE, HID), jnp.float32)
    z16 = jnp.zeros((ROWS_PER_TILE, 16), jnp.float32)
    ones16 = jnp.ones((SUB, 16), jnp.float32)
    b1 = b1_l.reshape(1, HID)
    b2 = b2_l.reshape(1, HID)

    p1, q1 = _tc_pre(x_pad, W1_l, W1_r, b1)
    part1, degp = _sc_agg_deg(p1, src2d, dst2d, z64, z16, ones16)
    p2, q2 = _tc_mid(part1, degp, q1, W2_l, W2_r, b2)
    part2 = _sc_agg(p2, src2d, dst2d, z64)
    z_pad = _tc_out(part2, degp, q2)
    return z_pad[:N_NODES]


# speedup vs baseline: 5.0439x; 1.0335x over previous
import functools

import jax
import jax.numpy as jnp
from jax import lax
from jax.experimental import pallas as pl
from jax.experimental.pallas import tpu as pltpu
from jax.experimental.pallas import tpu_sc as plsc

N_NODES = 10000
E = 320000
IN_DIM = 128
HID = 64

NC = 2
NS = 16
NW = NC * NS
SUB = 128
K = 4
CHUNK = SUB * K
CHUNKS_PER_W = 20
E_PAD = NW * CHUNK * CHUNKS_PER_W
IDXROWS_PER_W = E_PAD // SUB // NW
N_PAD = 10240
ROWS_PER_TILE = N_PAD // NS
DUMMY = N_NODES
RBLK = 512


def _tc_pre_body(x_ref, wl_ref, wr_ref, b_ref, p_ref, q_ref):
    xb = x_ref[...]
    p_ref[...] = jnp.dot(xb, wl_ref[...], preferred_element_type=jnp.float32)
    q_ref[...] = (jnp.dot(xb, wr_ref[...], preferred_element_type=jnp.float32)
                  + b_ref[...])


def _tc_pre(x_pad, W_l, W_r, b):
    grid = (N_PAD // RBLK,)
    return pl.pallas_call(
        _tc_pre_body,
        grid=grid,
        in_specs=[
            pl.BlockSpec((RBLK, IN_DIM), lambda i: (i, 0)),
            pl.BlockSpec((IN_DIM, HID), lambda i: (0, 0)),
            pl.BlockSpec((IN_DIM, HID), lambda i: (0, 0)),
            pl.BlockSpec((1, HID), lambda i: (0, 0)),
        ],
        out_specs=[
            pl.BlockSpec((RBLK, HID), lambda i: (i, 0)),
            pl.BlockSpec((RBLK, HID), lambda i: (i, 0)),
        ],
        out_shape=[
            jax.ShapeDtypeStruct((N_PAD, HID), jnp.float32),
            jax.ShapeDtypeStruct((N_PAD, HID), jnp.float32),
        ],
    )(x_pad, W_l, W_r, b)


def _tc_mid_body(part_ref, degp_ref, q1_ref, wl_ref, wr_ref, b_ref,
                 p2_ref, q2_ref):
    agg = part_ref[0] + part_ref[1]
    deg = degp_ref[0, :, 0] + degp_ref[1, :, 0]
    mean = agg / jnp.maximum(deg, 1.0)[:, None]
    h = jnp.maximum(mean + q1_ref[...], 0.0)
    p2_ref[...] = jnp.dot(h, wl_ref[...], preferred_element_type=jnp.float32)
    q2_ref[...] = (jnp.dot(h, wr_ref[...], preferred_element_type=jnp.float32)
                   + b_ref[...])


def _tc_mid(part, degp, q1, W_l, W_r, b):
    grid = (N_PAD // RBLK,)
    return pl.pallas_call(
        _tc_mid_body,
        grid=grid,
        in_specs=[
            pl.BlockSpec((NC, RBLK, HID), lambda i: (0, i, 0)),
            pl.BlockSpec((NC, RBLK, 16), lambda i: (0, i, 0)),
            pl.BlockSpec((RBLK, HID), lambda i: (i, 0)),
            pl.BlockSpec((HID, HID), lambda i: (0, 0)),
            pl.BlockSpec((HID, HID), lambda i: (0, 0)),
            pl.BlockSpec((1, HID), lambda i: (0, 0)),
        ],
        out_specs=[
            pl.BlockSpec((RBLK, HID), lambda i: (i, 0)),
            pl.BlockSpec((RBLK, HID), lambda i: (i, 0)),
        ],
        out_shape=[
            jax.ShapeDtypeStruct((N_PAD, HID), jnp.float32),
            jax.ShapeDtypeStruct((N_PAD, HID), jnp.float32),
        ],
    )(part, degp, q1, W_l, W_r, b)


def _tc_out_body(part_ref, degp_ref, q2_ref, z_ref):
    agg = part_ref[0] + part_ref[1]
    deg = degp_ref[0, :, 0] + degp_ref[1, :, 0]
    z_ref[...] = agg / jnp.maximum(deg, 1.0)[:, None] + q2_ref[...]


def _tc_out(part, degp, q2):
    grid = (N_PAD // RBLK,)
    return pl.pallas_call(
        _tc_out_body,
        grid=grid,
        in_specs=[
            pl.BlockSpec((NC, RBLK, HID), lambda i: (0, i, 0)),
            pl.BlockSpec((NC, RBLK, 16), lambda i: (0, i, 0)),
            pl.BlockSpec((RBLK, HID), lambda i: (i, 0)),
        ],
        out_specs=pl.BlockSpec((RBLK, HID), lambda i: (i, 0)),
        out_shape=jax.ShapeDtypeStruct((N_PAD, HID), jnp.float32),
    )(part, degp, q2)


def _make_sc_agg(with_deg):
    mesh = plsc.VectorSubcoreMesh(core_axis_name="c", subcore_axis_name="s")
    out_type = [jax.ShapeDtypeStruct((NC, N_PAD, HID), jnp.float32)]
    scratch = [
        pltpu.VMEM((K, 2, SUB), jnp.int32),
        pltpu.VMEM((K, 2, SUB), jnp.int32),
        pltpu.VMEM((CHUNK, HID), jnp.float32),
        pltpu.VMEM((CHUNK, HID), jnp.float32),
        pltpu.VMEM_SHARED((N_PAD, HID), jnp.float32),
        pltpu.SemaphoreType.DMA,
        pltpu.SemaphoreType.DMA,
        pltpu.SemaphoreType.DMA,
        pltpu.SemaphoreType.DMA,
    ]
    if with_deg:
        out_type.append(jax.ShapeDtypeStruct((NC, N_PAD, 16), jnp.float32))
        scratch += [
            pltpu.VMEM((SUB, 16), jnp.float32),
            pltpu.VMEM_SHARED((N_PAD, 16), jnp.float32),
        ]

    @functools.partial(
        pl.kernel,
        mesh=mesh,
        out_type=out_type,
        scratch_types=scratch,
        compiler_params=pltpu.CompilerParams(use_tc_tiling_on_sc=False),
    )
    def k(*refs):
        if with_deg:
            (table_h, ei_h, z64_h, z16_h, ones_h, part_out, deg_out,
             idx0, idx1, rows0, rows1, acc, sg0, sg1, ss0, ss1,
             ones_v, dacc) = refs
        else:
            (table_h, ei_h, z64_h, part_out,
             idx0, idx1, rows0, rows1, acc, sg0, sg1, ss0, ss1) = refs
        c = lax.axis_index("c")
        s = lax.axis_index("s")
        w = s * NC + c
        tile_rows = pl.ds(s * ROWS_PER_TILE, ROWS_PER_TILE)
        pltpu.sync_copy(z64_h, acc.at[tile_rows])
        if with_deg:
            pltpu.sync_copy(z16_h, dacc.at[tile_rows])
            pltpu.sync_copy(ones_h, ones_v)
        plsc.subcore_barrier()

        idx = (idx0, idx1)
        rows = (rows0, rows1)
        sg = (sg0, sg1)
        ss = (ss0, ss1)

        def copy_idx(b, t):
            row0 = w * IDXROWS_PER_W + t * K
            pltpu.sync_copy(ei_h.at[pl.ds(row0, K)], idx[b])

        def issue_gathers(b):
            return [
                pltpu.async_copy(table_h.at[idx[b].at[j, 0]],
                                 rows[b].at[pl.ds(j * SUB, SUB)], sg[b])
                for j in range(K)
            ]

        def issue_scatters(b):
            ds = []
            for j in range(K):
                ds.append(pltpu.async_copy(rows[b].at[pl.ds(j * SUB, SUB)],
                                           acc.at[idx[b].at[j, 1]], ss[b],
                                           add=True))
                if with_deg:
                    ds.append(pltpu.async_copy(ones_v,
                                               dacc.at[idx[b].at[j, 1]],
                                               ss[b], add=True))
            return ds

        copy_idx(0, 0)
        dg = [None, None]
        dsc = [None, None]
        dg[0] = issue_gathers(0)
        for t in range(CHUNKS_PER_W):
            b = t % 2
            nb = 1 - b
            if t + 1 < CHUNKS_PER_W:
                if dsc[nb] is not None:
                    for d in dsc[nb]:
                        d.wait()
                copy_idx(nb, t + 1)
                dg[nb] = issue_gathers(nb)
            for d in dg[b]:
                d.wait()
            dsc[b] = issue_scatters(b)
        for bb in (0, 1):
            for d in dsc[bb]:
                d.wait()
        plsc.subcore_barrier()
        pltpu.sync_copy(acc.at[tile_rows], part_out.at[c].at[tile_rows])
        if with_deg:
            pltpu.sync_copy(dacc.at[tile_rows], deg_out.at[c].at[tile_rows])

    return k


_sc_agg_deg = _make_sc_agg(True)
_sc_agg = _make_sc_agg(False)


def kernel(x, edge_index, W1_l, b1_l, W1_r, W2_l, b2_l, W2_r):
    src = edge_index[0].astype(jnp.int32)
    dst = edge_index[1].astype(jnp.int32)
    pad = E_PAD - E
    fill = jnp.full((pad,), DUMMY, jnp.int32)
    src2d = jnp.concatenate([src, fill]).reshape(E_PAD // SUB, SUB)
    dst2d = jnp.concatenate([dst, fill]).reshape(E_PAD // SUB, SUB)
    ei2 = jnp.stack([src2d, dst2d], axis=1)
    x_pad = jnp.pad(x, ((0, N_PAD - N_NODES), (0, 0)))
    z64 = jnp.zeros((ROWS_PER_TILE, HID), jnp.float32)
    z16 = jnp.zeros((ROWS_PER_TILE, 16), jnp.float32)
    ones16 = jnp.ones((SUB, 16), jnp.float32)
    b1 = b1_l.reshape(1, HID)
    b2 = b2_l.reshape(1, HID)

    p1, q1 = _tc_pre(x_pad, W1_l, W1_r, b1)
    part1, degp = _sc_agg_deg(p1, ei2, z64, z16, ones16)
    p2, q2 = _tc_mid(part1, degp, q1, W2_l, W2_r, b2)
    (part2,) = _sc_agg(p2, ei2, z64)
    z_pad = _tc_out(part2, degp, q2)
    return z_pad[:N_NODES]


# speedup vs baseline: 14.1318x; 2.8018x over previous
import functools

import jax
import jax.numpy as jnp
from jax import lax
from jax.experimental import pallas as pl
from jax.experimental.pallas import tpu as pltpu
from jax.experimental.pallas import tpu_sc as plsc

N_NODES = 10000
E = 320000
IN_DIM = 128
HID = 64

NC = 2
NS = 16
NW = NC * NS
SUB = 128
K = 4
CHUNK = SUB * K
CHUNKS_PER_W = 20
E_PAD = NW * CHUNK * CHUNKS_PER_W
IDXROWS_PER_W = E_PAD // SUB // NW
N_PAD = 10240
ROWS_PER_TILE = N_PAD // NS
DUMMY = N_NODES
RBLK = 512
VDT = jnp.bfloat16


def _tc_pre_body(x_ref, wl_ref, wr_ref, b_ref, p_ref, q_ref):
    xb = x_ref[...]
    p_ref[...] = jnp.dot(
        xb, wl_ref[...], preferred_element_type=jnp.float32).astype(VDT)
    q_ref[...] = (jnp.dot(xb, wr_ref[...], preferred_element_type=jnp.float32)
                  + b_ref[...])


def _tc_pre(x_pad, W_l, W_r, b):
    grid = (N_PAD // RBLK,)
    return pl.pallas_call(
        _tc_pre_body,
        grid=grid,
        in_specs=[
            pl.BlockSpec((RBLK, IN_DIM), lambda i: (i, 0)),
            pl.BlockSpec((IN_DIM, HID), lambda i: (0, 0)),
            pl.BlockSpec((IN_DIM, HID), lambda i: (0, 0)),
            pl.BlockSpec((1, HID), lambda i: (0, 0)),
        ],
        out_specs=[
            pl.BlockSpec((RBLK, HID), lambda i: (i, 0)),
            pl.BlockSpec((RBLK, HID), lambda i: (i, 0)),
        ],
        out_shape=[
            jax.ShapeDtypeStruct((N_PAD, HID), VDT),
            jax.ShapeDtypeStruct((N_PAD, HID), jnp.float32),
        ],
    )(x_pad, W_l, W_r, b)


def _tc_mid_body(part_ref, degp_ref, q1_ref, wl_ref, wr_ref, b_ref,
                 p2_ref, q2_ref):
    agg = (part_ref[0].astype(jnp.float32)
           + part_ref[1].astype(jnp.float32))
    deg = degp_ref[0, :, 0] + degp_ref[1, :, 0]
    mean = agg / jnp.maximum(deg, 1.0)[:, None]
    h = jnp.maximum(mean + q1_ref[...], 0.0)
    p2_ref[...] = jnp.dot(
        h, wl_ref[...], preferred_element_type=jnp.float32).astype(VDT)
    q2_ref[...] = (jnp.dot(h, wr_ref[...], preferred_element_type=jnp.float32)
                   + b_ref[...])


def _tc_mid(part, degp, q1, W_l, W_r, b):
    grid = (N_PAD // RBLK,)
    return pl.pallas_call(
        _tc_mid_body,
        grid=grid,
        in_specs=[
            pl.BlockSpec((NC, RBLK, HID), lambda i: (0, i, 0)),
            pl.BlockSpec((NC, RBLK, 16), lambda i: (0, i, 0)),
            pl.BlockSpec((RBLK, HID), lambda i: (i, 0)),
            pl.BlockSpec((HID, HID), lambda i: (0, 0)),
            pl.BlockSpec((HID, HID), lambda i: (0, 0)),
            pl.BlockSpec((1, HID), lambda i: (0, 0)),
        ],
        out_specs=[
            pl.BlockSpec((RBLK, HID), lambda i: (i, 0)),
            pl.BlockSpec((RBLK, HID), lambda i: (i, 0)),
        ],
        out_shape=[
            jax.ShapeDtypeStruct((N_PAD, HID), VDT),
            jax.ShapeDtypeStruct((N_PAD, HID), jnp.float32),
        ],
    )(part, degp, q1, W_l, W_r, b)


def _tc_out_body(part_ref, degp_ref, q2_ref, z_ref):
    agg = (part_ref[0].astype(jnp.float32)
           + part_ref[1].astype(jnp.float32))
    deg = degp_ref[0, :, 0] + degp_ref[1, :, 0]
    z_ref[...] = agg / jnp.maximum(deg, 1.0)[:, None] + q2_ref[...]


def _tc_out(part, degp, q2):
    grid = (N_PAD // RBLK,)
    return pl.pallas_call(
        _tc_out_body,
        grid=grid,
        in_specs=[
            pl.BlockSpec((NC, RBLK, HID), lambda i: (0, i, 0)),
            pl.BlockSpec((NC, RBLK, 16), lambda i: (0, i, 0)),
            pl.BlockSpec((RBLK, HID), lambda i: (i, 0)),
        ],
        out_specs=pl.BlockSpec((RBLK, HID), lambda i: (i, 0)),
        out_shape=jax.ShapeDtypeStruct((N_PAD, HID), jnp.float32),
    )(part, degp, q2)


def _make_sc_agg(with_deg):
    mesh = plsc.VectorSubcoreMesh(core_axis_name="c", subcore_axis_name="s")
    out_type = [jax.ShapeDtypeStruct((NC, N_PAD, HID), VDT)]
    scratch = [
        pltpu.VMEM((K, 2, SUB), jnp.int32),
        pltpu.VMEM((K, 2, SUB), jnp.int32),
        pltpu.VMEM((CHUNK, HID), VDT),
        pltpu.VMEM((CHUNK, HID), VDT),
        pltpu.VMEM_SHARED((N_PAD, HID), VDT),
        pltpu.VMEM_SHARED((N_PAD, HID), VDT),
        pltpu.SemaphoreType.DMA,
        pltpu.SemaphoreType.DMA,
        pltpu.SemaphoreType.DMA,
        pltpu.SemaphoreType.DMA,
    ]
    if with_deg:
        out_type.append(jax.ShapeDtypeStruct((NC, N_PAD, 16), jnp.float32))
        scratch += [
            pltpu.VMEM((SUB, 16), jnp.float32),
            pltpu.VMEM_SHARED((N_PAD, 16), jnp.float32),
        ]

    @functools.partial(
        pl.kernel,
        mesh=mesh,
        out_type=out_type,
        scratch_types=scratch,
        compiler_params=pltpu.CompilerParams(use_tc_tiling_on_sc=False),
    )
    def k(*refs):
        if with_deg:
            (table_h, ei_h, z64_h, z16_h, ones_h, part_out, deg_out,
             idx0, idx1, rows0, rows1, acc, table_sp, sg0, sg1, ss0, ss1,
             ones_v, dacc) = refs
        else:
            (table_h, ei_h, z64_h, part_out,
             idx0, idx1, rows0, rows1, acc, table_sp, sg0, sg1, ss0,
             ss1) = refs
        c = lax.axis_index("c")
        s = lax.axis_index("s")
        w = s * NC + c
        tile_rows = pl.ds(s * ROWS_PER_TILE, ROWS_PER_TILE)
        pltpu.sync_copy(z64_h, acc.at[tile_rows])
        pltpu.sync_copy(table_h.at[tile_rows], table_sp.at[tile_rows])
        if with_deg:
            pltpu.sync_copy(z16_h, dacc.at[tile_rows])
            pltpu.sync_copy(ones_h, ones_v)
        plsc.subcore_barrier()

        idx = (idx0, idx1)
        rows = (rows0, rows1)
        sg = (sg0, sg1)
        ss = (ss0, ss1)

        def copy_idx(b, t):
            row0 = w * IDXROWS_PER_W + t * K
            pltpu.sync_copy(ei_h.at[pl.ds(row0, K)], idx[b])

        def issue_gathers(b):
            return [
                pltpu.async_copy(table_sp.at[idx[b].at[j, 0]],
                                 rows[b].at[pl.ds(j * SUB, SUB)], sg[b])
                for j in range(K)
            ]

        def issue_scatters(b):
            ds = []
            for j in range(K):
                ds.append(pltpu.async_copy(rows[b].at[pl.ds(j * SUB, SUB)],
                                           acc.at[idx[b].at[j, 1]], ss[b],
                                           add=True))
                if with_deg:
                    ds.append(pltpu.async_copy(ones_v,
                                               dacc.at[idx[b].at[j, 1]],
                                               ss[b], add=True))
            return ds

        copy_idx(0, 0)
        dg = [None, None]
        dsc = [None, None]
        dg[0] = issue_gathers(0)
        for t in range(CHUNKS_PER_W):
            b = t % 2
            nb = 1 - b
            if t + 1 < CHUNKS_PER_W:
                if dsc[nb] is not None:
                    for d in dsc[nb]:
                        d.wait()
                copy_idx(nb, t + 1)
                dg[nb] = issue_gathers(nb)
            for d in dg[b]:
                d.wait()
            dsc[b] = issue_scatters(b)
        for bb in (0, 1):
            for d in dsc[bb]:
                d.wait()
        plsc.subcore_barrier()
        pltpu.sync_copy(acc.at[tile_rows], part_out.at[c].at[tile_rows])
        if with_deg:
            pltpu.sync_copy(dacc.at[tile_rows], deg_out.at[c].at[tile_rows])

    return k


_sc_agg_deg = _make_sc_agg(True)


def kernel(x, edge_index, W1_l, b1_l, W1_r, W2_l, b2_l, W2_r):
    src = edge_index[0].astype(jnp.int32)
    dst = edge_index[1].astype(jnp.int32)
    pad = E_PAD - E
    fill = jnp.full((pad,), DUMMY, jnp.int32)
    src2d = jnp.concatenate([src, fill]).reshape(E_PAD // SUB, SUB)
    dst2d = jnp.concatenate([dst, fill]).reshape(E_PAD // SUB, SUB)
    ei2 = jnp.stack([src2d, dst2d], axis=1)
    x_pad = jnp.pad(x, ((0, N_PAD - N_NODES), (0, 0)))
    z64 = jnp.zeros((ROWS_PER_TILE, HID), VDT)
    z16 = jnp.zeros((ROWS_PER_TILE, 16), jnp.float32)
    ones16 = jnp.ones((SUB, 16), jnp.float32)
    b1 = b1_l.reshape(1, HID)
    b2 = b2_l.reshape(1, HID)

    p1, q1 = _tc_pre(x_pad, W1_l, W1_r, b1)
    part1, degp = _sc_agg_deg(p1, ei2, z64, z16, ones16)
    p2, q2 = _tc_mid(part1, degp, q1, W2_l, W2_r, b2)
    part2, _ = _sc_agg_deg(p2, ei2, z64, z16, ones16)
    z_pad = _tc_out(part2, degp, q2)
    return z_pad[:N_NODES]


# speedup vs baseline: 14.7782x; 1.0457x over previous
import functools

import jax
import jax.numpy as jnp
from jax import lax
from jax.experimental import pallas as pl
from jax.experimental.pallas import tpu as pltpu
from jax.experimental.pallas import tpu_sc as plsc

N_NODES = 10000
E = 320000
IN_DIM = 128
HID = 64

NC = 2
NS = 16
NW = NC * NS
SUB = 128
K = 8
CHUNK = SUB * K
CHUNKS_PER_W = 10
DW = 8
E_PAD = NW * CHUNK * CHUNKS_PER_W
IDXROWS_PER_W = E_PAD // SUB // NW
N_PAD = 10240
ROWS_PER_TILE = N_PAD // NS
DUMMY = N_NODES
RBLK = 512
VDT = jnp.bfloat16


def _tc_pre_body(x_ref, wl_ref, wr_ref, b_ref, p_ref, q_ref):
    xb = x_ref[...]
    p_ref[...] = jnp.dot(
        xb, wl_ref[...], preferred_element_type=jnp.float32).astype(VDT)
    q_ref[...] = (jnp.dot(xb, wr_ref[...], preferred_element_type=jnp.float32)
                  + b_ref[...])


def _tc_pre(x_pad, W_l, W_r, b):
    grid = (N_PAD // RBLK,)
    return pl.pallas_call(
        _tc_pre_body,
        grid=grid,
        in_specs=[
            pl.BlockSpec((RBLK, IN_DIM), lambda i: (i, 0)),
            pl.BlockSpec((IN_DIM, HID), lambda i: (0, 0)),
            pl.BlockSpec((IN_DIM, HID), lambda i: (0, 0)),
            pl.BlockSpec((1, HID), lambda i: (0, 0)),
        ],
        out_specs=[
            pl.BlockSpec((RBLK, HID), lambda i: (i, 0)),
            pl.BlockSpec((RBLK, HID), lambda i: (i, 0)),
        ],
        out_shape=[
            jax.ShapeDtypeStruct((N_PAD, HID), VDT),
            jax.ShapeDtypeStruct((N_PAD, HID), jnp.float32),
        ],
    )(x_pad, W_l, W_r, b)


def _tc_mid_body(part_ref, degp_ref, q1_ref, wl_ref, wr_ref, b_ref,
                 p2_ref, q2_ref):
    agg = (part_ref[0].astype(jnp.float32)
           + part_ref[1].astype(jnp.float32))
    deg = degp_ref[0, :, 0] + degp_ref[1, :, 0]
    mean = agg / jnp.maximum(deg, 1.0)[:, None]
    h = jnp.maximum(mean + q1_ref[...], 0.0)
    p2_ref[...] = jnp.dot(
        h, wl_ref[...], preferred_element_type=jnp.float32).astype(VDT)
    q2_ref[...] = (jnp.dot(h, wr_ref[...], preferred_element_type=jnp.float32)
                   + b_ref[...])


def _tc_mid(part, degp, q1, W_l, W_r, b):
    grid = (N_PAD // RBLK,)
    return pl.pallas_call(
        _tc_mid_body,
        grid=grid,
        in_specs=[
            pl.BlockSpec((NC, RBLK, HID), lambda i: (0, i, 0)),
            pl.BlockSpec((NC, RBLK, DW), lambda i: (0, i, 0)),
            pl.BlockSpec((RBLK, HID), lambda i: (i, 0)),
            pl.BlockSpec((HID, HID), lambda i: (0, 0)),
            pl.BlockSpec((HID, HID), lambda i: (0, 0)),
            pl.BlockSpec((1, HID), lambda i: (0, 0)),
        ],
        out_specs=[
            pl.BlockSpec((RBLK, HID), lambda i: (i, 0)),
            pl.BlockSpec((RBLK, HID), lambda i: (i, 0)),
        ],
        out_shape=[
            jax.ShapeDtypeStruct((N_PAD, HID), VDT),
            jax.ShapeDtypeStruct((N_PAD, HID), jnp.float32),
        ],
    )(part, degp, q1, W_l, W_r, b)


def _tc_out_body(part_ref, degp_ref, q2_ref, z_ref):
    agg = (part_ref[0].astype(jnp.float32)
           + part_ref[1].astype(jnp.float32))
    deg = degp_ref[0, :, 0] + degp_ref[1, :, 0]
    z_ref[...] = agg / jnp.maximum(deg, 1.0)[:, None] + q2_ref[...]


def _tc_out(part, degp, q2):
    grid = (N_PAD // RBLK,)
    return pl.pallas_call(
        _tc_out_body,
        grid=grid,
        in_specs=[
            pl.BlockSpec((NC, RBLK, HID), lambda i: (0, i, 0)),
            pl.BlockSpec((NC, RBLK, DW), lambda i: (0, i, 0)),
            pl.BlockSpec((RBLK, HID), lambda i: (i, 0)),
        ],
        out_specs=pl.BlockSpec((RBLK, HID), lambda i: (i, 0)),
        out_shape=jax.ShapeDtypeStruct((N_PAD, HID), jnp.float32),
    )(part, degp, q2)


def _make_sc_agg(with_deg):
    mesh = plsc.VectorSubcoreMesh(core_axis_name="c", subcore_axis_name="s")
    out_type = [jax.ShapeDtypeStruct((NC, N_PAD, HID), VDT)]
    scratch = [
        pltpu.VMEM((K, 2, SUB), jnp.int32),
        pltpu.VMEM((K, 2, SUB), jnp.int32),
        pltpu.VMEM((CHUNK, HID), VDT),
        pltpu.VMEM((CHUNK, HID), VDT),
        pltpu.VMEM_SHARED((N_PAD, HID), VDT),
        pltpu.VMEM_SHARED((N_PAD, HID), VDT),
        pltpu.SemaphoreType.DMA,
        pltpu.SemaphoreType.DMA,
        pltpu.SemaphoreType.DMA,
        pltpu.SemaphoreType.DMA,
    ]
    if with_deg:
        out_type.append(jax.ShapeDtypeStruct((NC, N_PAD, DW), jnp.float32))
        scratch += [
            pltpu.VMEM((SUB, DW), jnp.float32),
            pltpu.VMEM_SHARED((N_PAD, DW), jnp.float32),
        ]

    @functools.partial(
        pl.kernel,
        mesh=mesh,
        out_type=out_type,
        scratch_types=scratch,
        compiler_params=pltpu.CompilerParams(use_tc_tiling_on_sc=False),
    )
    def k(*refs):
        if with_deg:
            (table_h, ei_h, z64_h, z16_h, ones_h, part_out, deg_out,
             idx0, idx1, rows0, rows1, acc, table_sp, sg0, sg1, ss0, ss1,
             ones_v, dacc) = refs
        else:
            (table_h, ei_h, z64_h, part_out,
             idx0, idx1, rows0, rows1, acc, table_sp, sg0, sg1, ss0,
             ss1) = refs
        c = lax.axis_index("c")
        s = lax.axis_index("s")
        w = s * NC + c
        tile_rows = pl.ds(s * ROWS_PER_TILE, ROWS_PER_TILE)
        pltpu.sync_copy(z64_h, acc.at[tile_rows])
        pltpu.sync_copy(table_h.at[tile_rows], table_sp.at[tile_rows])
        if with_deg:
            pltpu.sync_copy(z16_h, dacc.at[tile_rows])
            pltpu.sync_copy(ones_h, ones_v)
        plsc.subcore_barrier()

        idx = (idx0, idx1)
        rows = (rows0, rows1)
        sg = (sg0, sg1)
        ss = (ss0, ss1)

        def copy_idx(b, t):
            row0 = w * IDXROWS_PER_W + t * K
            pltpu.sync_copy(ei_h.at[pl.ds(row0, K)], idx[b])

        def issue_gathers(b):
            return [
                pltpu.async_copy(table_sp.at[idx[b].at[j, 0]],
                                 rows[b].at[pl.ds(j * SUB, SUB)], sg[b])
                for j in range(K)
            ]

        def issue_scatters(b):
            ds = []
            for j in range(K):
                ds.append(pltpu.async_copy(rows[b].at[pl.ds(j * SUB, SUB)],
                                           acc.at[idx[b].at[j, 1]], ss[b],
                                           add=True))
                if with_deg:
                    ds.append(pltpu.async_copy(ones_v,
                                               dacc.at[idx[b].at[j, 1]],
                                               ss[b], add=True))
            return ds

        copy_idx(0, 0)
        dg = [None, None]
        dsc = [None, None]
        dg[0] = issue_gathers(0)
        for t in range(CHUNKS_PER_W):
            b = t % 2
            nb = 1 - b
            if t + 1 < CHUNKS_PER_W:
                if dsc[nb] is not None:
                    for d in dsc[nb]:
                        d.wait()
                copy_idx(nb, t + 1)
                dg[nb] = issue_gathers(nb)
            for d in dg[b]:
                d.wait()
            dsc[b] = issue_scatters(b)
        for bb in (0, 1):
            for d in dsc[bb]:
                d.wait()
        plsc.subcore_barrier()
        pltpu.sync_copy(acc.at[tile_rows], part_out.at[c].at[tile_rows])
        if with_deg:
            pltpu.sync_copy(dacc.at[tile_rows], deg_out.at[c].at[tile_rows])

    return k


_sc_agg_deg = _make_sc_agg(True)


def kernel(x, edge_index, W1_l, b1_l, W1_r, W2_l, b2_l, W2_r):
    src = edge_index[0].astype(jnp.int32)
    dst = edge_index[1].astype(jnp.int32)
    pad = E_PAD - E
    fill = jnp.full((pad,), DUMMY, jnp.int32)
    src2d = jnp.concatenate([src, fill]).reshape(E_PAD // SUB, SUB)
    dst2d = jnp.concatenate([dst, fill]).reshape(E_PAD // SUB, SUB)
    ei2 = jnp.stack([src2d, dst2d], axis=1)
    x_pad = jnp.pad(x, ((0, N_PAD - N_NODES), (0, 0)))
    z64 = jnp.zeros((ROWS_PER_TILE, HID), VDT)
    z16 = jnp.zeros((ROWS_PER_TILE, DW), jnp.float32)
    ones16 = jnp.ones((SUB, DW), jnp.float32)
    b1 = b1_l.reshape(1, HID)
    b2 = b2_l.reshape(1, HID)

    p1, q1 = _tc_pre(x_pad, W1_l, W1_r, b1)
    part1, degp = _sc_agg_deg(p1, ei2, z64, z16, ones16)
    p2, q2 = _tc_mid(part1, degp, q1, W2_l, W2_r, b2)
    part2, _ = _sc_agg_deg(p2, ei2, z64, z16, ones16)
    z_pad = _tc_out(part2, degp, q2)
    return z_pad[:N_NODES]


# speedup vs baseline: 15.2230x; 1.0301x over previous
import functools

import jax
import jax.numpy as jnp
from jax import lax
from jax.experimental import pallas as pl
from jax.experimental.pallas import tpu as pltpu
from jax.experimental.pallas import tpu_sc as plsc

N_NODES = 10000
E = 320000
IN_DIM = 128
HID = 64

NC = 2
NS = 16
NW = NC * NS
SUB = 128
K = 8
CHUNK = SUB * K
CHUNKS_PER_W = 10
DW = 8
E_PAD = NW * CHUNK * CHUNKS_PER_W
IDXROWS_PER_W = E_PAD // SUB // NW
N_PAD = 10240
ROWS_PER_TILE = N_PAD // NS
DUMMY = N_NODES
RBLK = 512
VDT = jnp.bfloat16


def _tc_pre_body(x_ref, wl_ref, wr_ref, b_ref, p_ref, q_ref):
    xb = x_ref[...]
    p_ref[...] = jnp.dot(
        xb, wl_ref[...], preferred_element_type=jnp.float32).astype(VDT)
    q_ref[...] = (jnp.dot(xb, wr_ref[...], preferred_element_type=jnp.float32)
                  + b_ref[...])


def _tc_pre(x_pad, W_l, W_r, b):
    grid = (N_PAD // RBLK,)
    return pl.pallas_call(
        _tc_pre_body,
        grid=grid,
        in_specs=[
            pl.BlockSpec((RBLK, IN_DIM), lambda i: (i, 0)),
            pl.BlockSpec((IN_DIM, HID), lambda i: (0, 0)),
            pl.BlockSpec((IN_DIM, HID), lambda i: (0, 0)),
            pl.BlockSpec((1, HID), lambda i: (0, 0)),
        ],
        out_specs=[
            pl.BlockSpec((RBLK, HID), lambda i: (i, 0)),
            pl.BlockSpec((RBLK, HID), lambda i: (i, 0)),
        ],
        out_shape=[
            jax.ShapeDtypeStruct((N_PAD, HID), VDT),
            jax.ShapeDtypeStruct((N_PAD, HID), jnp.float32),
        ],
    )(x_pad, W_l, W_r, b)


def _tc_mid_body(part_ref, degp_ref, q1_ref, wl_ref, wr_ref, b_ref,
                 p2_ref, q2_ref):
    agg = (part_ref[0].astype(jnp.float32)
           + part_ref[1].astype(jnp.float32))
    deg = degp_ref[0, :, 0] + degp_ref[1, :, 0]
    mean = agg / jnp.maximum(deg, 1.0)[:, None]
    h = jnp.maximum(mean + q1_ref[...], 0.0)
    p2_ref[...] = jnp.dot(
        h, wl_ref[...], preferred_element_type=jnp.float32).astype(VDT)
    q2_ref[...] = (jnp.dot(h, wr_ref[...], preferred_element_type=jnp.float32)
                   + b_ref[...])


def _tc_mid(part, degp, q1, W_l, W_r, b):
    grid = (N_PAD // RBLK,)
    return pl.pallas_call(
        _tc_mid_body,
        grid=grid,
        in_specs=[
            pl.BlockSpec((NC, RBLK, HID), lambda i: (0, i, 0)),
            pl.BlockSpec((NC, RBLK, DW), lambda i: (0, i, 0)),
            pl.BlockSpec((RBLK, HID), lambda i: (i, 0)),
            pl.BlockSpec((HID, HID), lambda i: (0, 0)),
            pl.BlockSpec((HID, HID), lambda i: (0, 0)),
            pl.BlockSpec((1, HID), lambda i: (0, 0)),
        ],
        out_specs=[
            pl.BlockSpec((RBLK, HID), lambda i: (i, 0)),
            pl.BlockSpec((RBLK, HID), lambda i: (i, 0)),
        ],
        out_shape=[
            jax.ShapeDtypeStruct((N_PAD, HID), VDT),
            jax.ShapeDtypeStruct((N_PAD, HID), jnp.float32),
        ],
    )(part, degp, q1, W_l, W_r, b)


def _tc_out_body(part_ref, degp_ref, q2_ref, z_ref):
    agg = (part_ref[0].astype(jnp.float32)
           + part_ref[1].astype(jnp.float32))
    deg = degp_ref[0, :, 0] + degp_ref[1, :, 0]
    z_ref[...] = agg / jnp.maximum(deg, 1.0)[:, None] + q2_ref[...]


def _tc_out(part, degp, q2):
    grid = (N_PAD // RBLK,)
    return pl.pallas_call(
        _tc_out_body,
        grid=grid,
        in_specs=[
            pl.BlockSpec((NC, RBLK, HID), lambda i: (0, i, 0)),
            pl.BlockSpec((NC, RBLK, DW), lambda i: (0, i, 0)),
            pl.BlockSpec((RBLK, HID), lambda i: (i, 0)),
        ],
        out_specs=pl.BlockSpec((RBLK, HID), lambda i: (i, 0)),
        out_shape=jax.ShapeDtypeStruct((N_PAD, HID), jnp.float32),
    )(part, degp, q2)


def _make_sc_agg(with_deg):
    mesh = plsc.VectorSubcoreMesh(core_axis_name="c", subcore_axis_name="s")
    out_type = [jax.ShapeDtypeStruct((NC, N_PAD, HID), VDT)]
    scratch = [
        pltpu.VMEM((K, 2, SUB), jnp.int32),
        pltpu.VMEM((K, 2, SUB), jnp.int32),
        pltpu.VMEM((CHUNK, HID), VDT),
        pltpu.VMEM((CHUNK, HID), VDT),
        pltpu.VMEM_SHARED((N_PAD, HID), VDT),
        pltpu.VMEM_SHARED((N_PAD, HID), VDT),
        pltpu.SemaphoreType.DMA,
        pltpu.SemaphoreType.DMA,
        pltpu.SemaphoreType.DMA,
        pltpu.SemaphoreType.DMA,
    ]
    if with_deg:
        out_type.append(jax.ShapeDtypeStruct((NC, N_PAD, DW), jnp.float32))
        scratch += [
            pltpu.VMEM((SUB, DW), jnp.float32),
            pltpu.VMEM_SHARED((N_PAD, DW), jnp.float32),
        ]

    @functools.partial(
        pl.kernel,
        mesh=mesh,
        out_type=out_type,
        scratch_types=scratch,
        compiler_params=pltpu.CompilerParams(use_tc_tiling_on_sc=False),
    )
    def k(*refs):
        if with_deg:
            (table_h, ei_h, z64_h, z16_h, ones_h, part_out, deg_out,
             idx0, idx1, rows0, rows1, acc, table_sp, sg0, sg1, ss0, ss1,
             ones_v, dacc) = refs
        else:
            (table_h, ei_h, z64_h, part_out,
             idx0, idx1, rows0, rows1, acc, table_sp, sg0, sg1, ss0,
             ss1) = refs
        c = lax.axis_index("c")
        s = lax.axis_index("s")
        w = s * NC + c
        tile_rows = pl.ds(s * ROWS_PER_TILE, ROWS_PER_TILE)
        pltpu.sync_copy(z64_h, acc.at[tile_rows])
        pltpu.sync_copy(table_h.at[tile_rows], table_sp.at[tile_rows])
        if with_deg:
            pltpu.sync_copy(z16_h, dacc.at[tile_rows])
            pltpu.sync_copy(ones_h, ones_v)
        plsc.subcore_barrier()

        idx = (idx0, idx1)
        rows = (rows0, rows1)
        sg = (sg0, sg1)
        ss = (ss0, ss1)

        def copy_idx(b, t):
            row0 = w * IDXROWS_PER_W + t * K
            pltpu.sync_copy(ei_h.at[pl.ds(row0, K)], idx[b])

        def issue_gathers(b):
            return [
                pltpu.async_copy(table_sp.at[idx[b].at[j, 0]],
                                 rows[b].at[pl.ds(j * SUB, SUB)], sg[b])
                for j in range(K)
            ]

        def issue_scatters(b):
            ds = []
            for j in range(K):
                ds.append(pltpu.async_copy(rows[b].at[pl.ds(j * SUB, SUB)],
                                           acc.at[idx[b].at[j, 1]], ss[b],
                                           add=True))
                if with_deg:
                    ds.append(pltpu.async_copy(ones_v,
                                               dacc.at[idx[b].at[j, 1]],
                                               ss[b], add=True))
            return ds

        copy_idx(0, 0)
        dg = [None, None]
        dsc = [None, None]
        dg[0] = issue_gathers(0)
        for t in range(CHUNKS_PER_W):
            b = t % 2
            nb = 1 - b
            if t + 1 < CHUNKS_PER_W:
                if dsc[nb] is not None:
                    for d in dsc[nb]:
                        d.wait()
                copy_idx(nb, t + 1)
                dg[nb] = issue_gathers(nb)
            for d in dg[b]:
                d.wait()
            dsc[b] = issue_scatters(b)
        for bb in (0, 1):
            for d in dsc[bb]:
                d.wait()
        plsc.subcore_barrier()
        pltpu.sync_copy(acc.at[tile_rows], part_out.at[c].at[tile_rows])
        if with_deg:
            pltpu.sync_copy(dacc.at[tile_rows], deg_out.at[c].at[tile_rows])

    return k


_sc_agg_deg = _make_sc_agg(True)
_sc_agg = _make_sc_agg(False)


def kernel(x, edge_index, W1_l, b1_l, W1_r, W2_l, b2_l, W2_r):
    src = edge_index[0].astype(jnp.int32)
    dst = edge_index[1].astype(jnp.int32)
    pad = E_PAD - E
    fill = jnp.full((pad,), DUMMY, jnp.int32)
    src2d = jnp.concatenate([src, fill]).reshape(E_PAD // SUB, SUB)
    dst2d = jnp.concatenate([dst, fill]).reshape(E_PAD // SUB, SUB)
    ei2 = jnp.stack([src2d, dst2d], axis=1)
    x_pad = jnp.pad(x, ((0, N_PAD - N_NODES), (0, 0)))
    z64 = jnp.zeros((ROWS_PER_TILE, HID), VDT)
    z16 = jnp.zeros((ROWS_PER_TILE, DW), jnp.float32)
    ones16 = jnp.ones((SUB, DW), jnp.float32)
    b1 = b1_l.reshape(1, HID)
    b2 = b2_l.reshape(1, HID)

    p1, q1 = _tc_pre(x_pad, W1_l, W1_r, b1)
    part1, degp = _sc_agg_deg(p1, ei2, z64, z16, ones16)
    p2, q2 = _tc_mid(part1, degp, q1, W2_l, W2_r, b2)
    (part2,) = _sc_agg(p2, ei2, z64)
    z_pad = _tc_out(part2, degp, q2)
    return z_pad[:N_NODES]


# speedup vs baseline: 15.2852x; 1.0041x over previous
import functools

import jax
import jax.numpy as jnp
from jax import lax
from jax.experimental import pallas as pl
from jax.experimental.pallas import tpu as pltpu
from jax.experimental.pallas import tpu_sc as plsc

N_NODES = 10000
E = 320000
IN_DIM = 128
HID = 64

NC = 2
NS = 16
NW = NC * NS
SUB = 128
K = 8
CHUNK = SUB * K
CHUNKS_PER_W = 10
DW = 8
E_PAD = NW * CHUNK * CHUNKS_PER_W
IDXROWS_PER_W = E_PAD // SUB // NW
N_PAD = 10240
ROWS_PER_TILE = N_PAD // NS
DUMMY = N_NODES
RBLK = 512
VDT = jnp.bfloat16


def _tc_pre_body(x_ref, wl_ref, wr_ref, b_ref, p_ref, q_ref):
    xb = x_ref[...]
    p_ref[...] = jnp.dot(
        xb, wl_ref[...], preferred_element_type=jnp.float32).astype(VDT)
    q_ref[...] = (jnp.dot(xb, wr_ref[...], preferred_element_type=jnp.float32)
                  + b_ref[...])


def _tc_pre(x_pad, W_l, W_r, b):
    grid = (N_PAD // RBLK,)
    return pl.pallas_call(
        _tc_pre_body,
        grid=grid,
        in_specs=[
            pl.BlockSpec((RBLK, IN_DIM), lambda i: (i, 0)),
            pl.BlockSpec((IN_DIM, HID), lambda i: (0, 0)),
            pl.BlockSpec((IN_DIM, HID), lambda i: (0, 0)),
            pl.BlockSpec((1, HID), lambda i: (0, 0)),
        ],
        out_specs=[
            pl.BlockSpec((RBLK, HID), lambda i: (i, 0)),
            pl.BlockSpec((RBLK, HID), lambda i: (i, 0)),
        ],
        out_shape=[
            jax.ShapeDtypeStruct((N_PAD, HID), VDT),
            jax.ShapeDtypeStruct((N_PAD, HID), jnp.float32),
        ],
    )(x_pad, W_l, W_r, b)


def _tc_mid_body(part_ref, degp_ref, q1_ref, wl_ref, wr_ref, b_ref,
                 p2_ref, q2_ref):
    agg = (part_ref[0].astype(jnp.float32)
           + part_ref[1].astype(jnp.float32))
    deg = degp_ref[0, :, 0] + degp_ref[1, :, 0]
    mean = agg / jnp.maximum(deg, 1.0)[:, None]
    h = jnp.maximum(mean + q1_ref[...], 0.0)
    p2_ref[...] = jnp.dot(
        h, wl_ref[...], preferred_element_type=jnp.float32).astype(VDT)
    q2_ref[...] = (jnp.dot(h, wr_ref[...], preferred_element_type=jnp.float32)
                   + b_ref[...])


def _tc_mid(part, degp, q1, W_l, W_r, b):
    grid = (N_PAD // RBLK,)
    return pl.pallas_call(
        _tc_mid_body,
        grid=grid,
        in_specs=[
            pl.BlockSpec((NC, RBLK, HID), lambda i: (0, i, 0)),
            pl.BlockSpec((NC, RBLK, DW), lambda i: (0, i, 0)),
            pl.BlockSpec((RBLK, HID), lambda i: (i, 0)),
            pl.BlockSpec((HID, HID), lambda i: (0, 0)),
            pl.BlockSpec((HID, HID), lambda i: (0, 0)),
            pl.BlockSpec((1, HID), lambda i: (0, 0)),
        ],
        out_specs=[
            pl.BlockSpec((RBLK, HID), lambda i: (i, 0)),
            pl.BlockSpec((RBLK, HID), lambda i: (i, 0)),
        ],
        out_shape=[
            jax.ShapeDtypeStruct((N_PAD, HID), VDT),
            jax.ShapeDtypeStruct((N_PAD, HID), jnp.float32),
        ],
    )(part, degp, q1, W_l, W_r, b)


def _tc_out_body(part_ref, degp_ref, q2_ref, z_ref):
    agg = (part_ref[0].astype(jnp.float32)
           + part_ref[1].astype(jnp.float32))
    deg = degp_ref[0, :, 0] + degp_ref[1, :, 0]
    z_ref[...] = agg / jnp.maximum(deg, 1.0)[:, None] + q2_ref[...]


def _tc_out(part, degp, q2):
    oblk = 400
    grid = (N_NODES // oblk,)
    return pl.pallas_call(
        _tc_out_body,
        grid=grid,
        in_specs=[
            pl.BlockSpec((NC, oblk, HID), lambda i: (0, i, 0)),
            pl.BlockSpec((NC, oblk, DW), lambda i: (0, i, 0)),
            pl.BlockSpec((oblk, HID), lambda i: (i, 0)),
        ],
        out_specs=pl.BlockSpec((oblk, HID), lambda i: (i, 0)),
        out_shape=jax.ShapeDtypeStruct((N_NODES, HID), jnp.float32),
    )(part, degp, q2)


def _make_sc_agg(with_deg):
    mesh = plsc.VectorSubcoreMesh(core_axis_name="c", subcore_axis_name="s")
    out_type = [jax.ShapeDtypeStruct((NC, N_PAD, HID), VDT)]
    scratch = [
        pltpu.VMEM((K, 2, SUB), jnp.int32),
        pltpu.VMEM((K, 2, SUB), jnp.int32),
        pltpu.VMEM((CHUNK, HID), VDT),
        pltpu.VMEM((CHUNK, HID), VDT),
        pltpu.VMEM_SHARED((N_PAD, HID), VDT),
        pltpu.VMEM_SHARED((N_PAD, HID), VDT),
        pltpu.SemaphoreType.DMA,
        pltpu.SemaphoreType.DMA,
        pltpu.SemaphoreType.DMA,
        pltpu.SemaphoreType.DMA,
    ]
    if with_deg:
        out_type.append(jax.ShapeDtypeStruct((NC, N_PAD, DW), jnp.float32))
        scratch += [
            pltpu.VMEM((SUB, DW), jnp.float32),
            pltpu.VMEM_SHARED((N_PAD, DW), jnp.float32),
        ]

    @functools.partial(
        pl.kernel,
        mesh=mesh,
        out_type=out_type,
        scratch_types=scratch,
        compiler_params=pltpu.CompilerParams(use_tc_tiling_on_sc=False),
    )
    def k(*refs):
        if with_deg:
            (table_h, ei_h, z64_h, z16_h, ones_h, part_out, deg_out,
             idx0, idx1, rows0, rows1, acc, table_sp, sg0, sg1, ss0, ss1,
             ones_v, dacc) = refs
        else:
            (table_h, ei_h, z64_h, part_out,
             idx0, idx1, rows0, rows1, acc, table_sp, sg0, sg1, ss0,
             ss1) = refs
        c = lax.axis_index("c")
        s = lax.axis_index("s")
        w = s * NC + c
        tile_rows = pl.ds(s * ROWS_PER_TILE, ROWS_PER_TILE)
        pltpu.sync_copy(z64_h, acc.at[tile_rows])
        pltpu.sync_copy(table_h.at[tile_rows], table_sp.at[tile_rows])
        if with_deg:
            pltpu.sync_copy(z16_h, dacc.at[tile_rows])
            pltpu.sync_copy(ones_h, ones_v)
        plsc.subcore_barrier()

        idx = (idx0, idx1)
        rows = (rows0, rows1)
        sg = (sg0, sg1)
        ss = (ss0, ss1)

        def copy_idx(b, t):
            row0 = w * IDXROWS_PER_W + t * K
            pltpu.sync_copy(ei_h.at[pl.ds(row0, K)], idx[b])

        def issue_gathers(b):
            return [
                pltpu.async_copy(table_sp.at[idx[b].at[j, 0]],
                                 rows[b].at[pl.ds(j * SUB, SUB)], sg[b])
                for j in range(K)
            ]

        def issue_scatters(b):
            ds = []
            for j in range(K):
                ds.append(pltpu.async_copy(rows[b].at[pl.ds(j * SUB, SUB)],
                                           acc.at[idx[b].at[j, 1]], ss[b],
                                           add=True))
                if with_deg:
                    ds.append(pltpu.async_copy(ones_v,
                                               dacc.at[idx[b].at[j, 1]],
                                               ss[b], add=True))
            return ds

        copy_idx(0, 0)
        dg = [None, None]
        dsc = [None, None]
        dg[0] = issue_gathers(0)
        for t in range(CHUNKS_PER_W):
            b = t % 2
            nb = 1 - b
            if t + 1 < CHUNKS_PER_W:
                if dsc[nb] is not None:
                    for d in dsc[nb]:
                        d.wait()
                copy_idx(nb, t + 1)
                dg[nb] = issue_gathers(nb)
            for d in dg[b]:
                d.wait()
            dsc[b] = issue_scatters(b)
        for bb in (0, 1):
            for d in dsc[bb]:
                d.wait()
        plsc.subcore_barrier()
        pltpu.sync_copy(acc.at[tile_rows], part_out.at[c].at[tile_rows])
        if with_deg:
            pltpu.sync_copy(dacc.at[tile_rows], deg_out.at[c].at[tile_rows])

    return k


_sc_agg_deg = _make_sc_agg(True)
_sc_agg = _make_sc_agg(False)


def kernel(x, edge_index, W1_l, b1_l, W1_r, W2_l, b2_l, W2_r):
    src = edge_index[0].astype(jnp.int32)
    dst = edge_index[1].astype(jnp.int32)
    pad = E_PAD - E
    fill = jnp.full((pad,), DUMMY, jnp.int32)
    src2d = jnp.concatenate([src, fill]).reshape(E_PAD // SUB, SUB)
    dst2d = jnp.concatenate([dst, fill]).reshape(E_PAD // SUB, SUB)
    ei2 = jnp.stack([src2d, dst2d], axis=1)
    z64 = jnp.zeros((ROWS_PER_TILE, HID), VDT)
    z16 = jnp.zeros((ROWS_PER_TILE, DW), jnp.float32)
    ones16 = jnp.ones((SUB, DW), jnp.float32)
    b1 = b1_l.reshape(1, HID)
    b2 = b2_l.reshape(1, HID)

    p1, q1 = _tc_pre(x, W1_l, W1_r, b1)
    part1, degp = _sc_agg_deg(p1, ei2, z64, z16, ones16)
    p2, q2 = _tc_mid(part1, degp, q1, W2_l, W2_r, b2)
    (part2,) = _sc_agg(p2, ei2, z64)
    return _tc_out(part2, degp, q2)


# speedup vs baseline: 15.5579x; 1.0178x over previous
import functools

import jax
import jax.numpy as jnp
from jax import lax
from jax.experimental import pallas as pl
from jax.experimental.pallas import tpu as pltpu
from jax.experimental.pallas import tpu_sc as plsc

N_NODES = 10000
E = 320000
IN_DIM = 128
HID = 64

NC = 2
NS = 16
NW = NC * NS
SUB = 128
K = 8
CHUNK = SUB * K
CHUNKS_PER_W = 10
DW = 8
E_PAD = NW * CHUNK * CHUNKS_PER_W
IDXROWS_PER_W = E_PAD // SUB // NW
N_PAD = 10240
ROWS_PER_TILE = N_PAD // NS
DUMMY = N_NODES
RBLK = 512
VDT = jnp.bfloat16


def _tc_pre_body(x_ref, wl_ref, wr_ref, b_ref, p_ref, q_ref):
    xb = x_ref[...]
    p_ref[...] = jnp.dot(
        xb, wl_ref[...], preferred_element_type=jnp.float32).astype(VDT)
    q_ref[...] = (jnp.dot(xb, wr_ref[...], preferred_element_type=jnp.float32)
                  + b_ref[...])


def _tc_pre(x_pad, W_l, W_r, b):
    grid = (N_PAD // RBLK,)
    return pl.pallas_call(
        _tc_pre_body,
        grid=grid,
        in_specs=[
            pl.BlockSpec((RBLK, IN_DIM), lambda i: (i, 0)),
            pl.BlockSpec((IN_DIM, HID), lambda i: (0, 0)),
            pl.BlockSpec((IN_DIM, HID), lambda i: (0, 0)),
            pl.BlockSpec((1, HID), lambda i: (0, 0)),
        ],
        out_specs=[
            pl.BlockSpec((RBLK, HID), lambda i: (i, 0)),
            pl.BlockSpec((RBLK, HID), lambda i: (i, 0)),
        ],
        out_shape=[
            jax.ShapeDtypeStruct((N_PAD, HID), VDT),
            jax.ShapeDtypeStruct((N_PAD, HID), jnp.float32),
        ],
    )(x_pad, W_l, W_r, b)


def _tc_mid_body(part_ref, degp_ref, q1_ref, wl_ref, wr_ref, b_ref,
                 p2_ref, q2_ref):
    agg = (part_ref[0].astype(jnp.float32)
           + part_ref[1].astype(jnp.float32))
    deg = degp_ref[0, :, 0] + degp_ref[1, :, 0]
    mean = agg / jnp.maximum(deg, 1.0)[:, None]
    h = jnp.maximum(mean + q1_ref[...], 0.0)
    p2_ref[...] = jnp.dot(
        h, wl_ref[...], preferred_element_type=jnp.float32).astype(VDT)
    q2_ref[...] = (jnp.dot(h, wr_ref[...], preferred_element_type=jnp.float32)
                   + b_ref[...])


def _tc_mid(part, degp, q1, W_l, W_r, b):
    grid = (N_PAD // RBLK,)
    return pl.pallas_call(
        _tc_mid_body,
        grid=grid,
        in_specs=[
            pl.BlockSpec((NC, RBLK, HID), lambda i: (0, i, 0)),
            pl.BlockSpec((NC, RBLK, DW), lambda i: (0, i, 0)),
            pl.BlockSpec((RBLK, HID), lambda i: (i, 0)),
            pl.BlockSpec((HID, HID), lambda i: (0, 0)),
            pl.BlockSpec((HID, HID), lambda i: (0, 0)),
            pl.BlockSpec((1, HID), lambda i: (0, 0)),
        ],
        out_specs=[
            pl.BlockSpec((RBLK, HID), lambda i: (i, 0)),
            pl.BlockSpec((RBLK, HID), lambda i: (i, 0)),
        ],
        out_shape=[
            jax.ShapeDtypeStruct((N_PAD, HID), VDT),
            jax.ShapeDtypeStruct((N_PAD, HID), jnp.float32),
        ],
    )(part, degp, q1, W_l, W_r, b)


def _tc_out_body(part_ref, degp_ref, q2_ref, z_ref):
    agg = (part_ref[0].astype(jnp.float32)
           + part_ref[1].astype(jnp.float32))
    deg = degp_ref[0, :, 0] + degp_ref[1, :, 0]
    z_ref[...] = agg / jnp.maximum(deg, 1.0)[:, None] + q2_ref[...]


def _tc_out(part, degp, q2):
    oblk = 400
    grid = (N_NODES // oblk,)
    return pl.pallas_call(
        _tc_out_body,
        grid=grid,
        in_specs=[
            pl.BlockSpec((NC, oblk, HID), lambda i: (0, i, 0)),
            pl.BlockSpec((NC, oblk, DW), lambda i: (0, i, 0)),
            pl.BlockSpec((oblk, HID), lambda i: (i, 0)),
        ],
        out_specs=pl.BlockSpec((oblk, HID), lambda i: (i, 0)),
        out_shape=jax.ShapeDtypeStruct((N_NODES, HID), jnp.float32),
    )(part, degp, q2)


def _make_sc_agg(with_deg):
    mesh = plsc.VectorSubcoreMesh(core_axis_name="c", subcore_axis_name="s")
    out_type = [jax.ShapeDtypeStruct((NC, N_PAD, HID), VDT)]
    scratch = [
        pltpu.VMEM((K, 2, SUB), jnp.int32),
        pltpu.VMEM((K, 2, SUB), jnp.int32),
        pltpu.VMEM((CHUNK, HID), VDT),
        pltpu.VMEM((CHUNK, HID), VDT),
        pltpu.VMEM_SHARED((N_PAD, HID), VDT),
        pltpu.VMEM_SHARED((N_PAD, HID), VDT),
        pltpu.SemaphoreType.DMA,
        pltpu.SemaphoreType.DMA,
        pltpu.SemaphoreType.DMA,
        pltpu.SemaphoreType.DMA,
    ]
    if with_deg:
        out_type.append(jax.ShapeDtypeStruct((NC, N_PAD, DW), jnp.float32))
        scratch += [
            pltpu.VMEM((SUB, DW), jnp.float32),
            pltpu.VMEM_SHARED((N_PAD, DW), jnp.float32),
        ]

    @functools.partial(
        pl.kernel,
        mesh=mesh,
        out_type=out_type,
        scratch_types=scratch,
        compiler_params=pltpu.CompilerParams(use_tc_tiling_on_sc=False),
    )
    def k(*refs):
        if with_deg:
            (table_h, ei_h, z64_h, z16_h, ones_h, part_out, deg_out,
             idx0, idx1, rows0, rows1, acc, table_sp, sg0, sg1, ss0, ss1,
             ones_v, dacc) = refs
        else:
            (table_h, ei_h, z64_h, part_out,
             idx0, idx1, rows0, rows1, acc, table_sp, sg0, sg1, ss0,
             ss1) = refs
        c = lax.axis_index("c")
        s = lax.axis_index("s")
        w = s * NC + c
        tile_rows = pl.ds(s * ROWS_PER_TILE, ROWS_PER_TILE)
        pro = [
            pltpu.async_copy(z64_h, acc.at[tile_rows], sg0),
            pltpu.async_copy(table_h.at[tile_rows], table_sp.at[tile_rows],
                             sg1),
        ]
        if with_deg:
            pro.append(pltpu.async_copy(z16_h, dacc.at[tile_rows], ss0))
            pro.append(pltpu.async_copy(ones_h, ones_v, ss1))
        for d in pro:
            d.wait()
        plsc.subcore_barrier()

        idx = (idx0, idx1)
        rows = (rows0, rows1)
        sg = (sg0, sg1)
        ss = (ss0, ss1)

        def copy_idx(b, t):
            row0 = w * IDXROWS_PER_W + t * K
            pltpu.sync_copy(ei_h.at[pl.ds(row0, K)], idx[b])

        def issue_gathers(b):
            return [
                pltpu.async_copy(table_sp.at[idx[b].at[j, 0]],
                                 rows[b].at[pl.ds(j * SUB, SUB)], sg[b])
                for j in range(K)
            ]

        def issue_scatters(b):
            ds = []
            for j in range(K):
                ds.append(pltpu.async_copy(rows[b].at[pl.ds(j * SUB, SUB)],
                                           acc.at[idx[b].at[j, 1]], ss[b],
                                           add=True))
                if with_deg:
                    ds.append(pltpu.async_copy(ones_v,
                                               dacc.at[idx[b].at[j, 1]],
                                               ss[b], add=True))
            return ds

        copy_idx(0, 0)
        dg = [None, None]
        dsc = [None, None]
        dg[0] = issue_gathers(0)
        for t in range(CHUNKS_PER_W):
            b = t % 2
            nb = 1 - b
            if t + 1 < CHUNKS_PER_W:
                if dsc[nb] is not None:
                    for d in dsc[nb]:
                        d.wait()
                copy_idx(nb, t + 1)
                dg[nb] = issue_gathers(nb)
            for d in dg[b]:
                d.wait()
            dsc[b] = issue_scatters(b)
        for bb in (0, 1):
            for d in dsc[bb]:
                d.wait()
        plsc.subcore_barrier()
        epi = [pltpu.async_copy(acc.at[tile_rows],
                                part_out.at[c].at[tile_rows], sg0)]
        if with_deg:
            epi.append(pltpu.async_copy(dacc.at[tile_rows],
                                        deg_out.at[c].at[tile_rows], sg1))
        for d in epi:
            d.wait()

    return k


_sc_agg_deg = _make_sc_agg(True)
_sc_agg = _make_sc_agg(False)


def kernel(x, edge_index, W1_l, b1_l, W1_r, W2_l, b2_l, W2_r):
    src = edge_index[0].astype(jnp.int32)
    dst = edge_index[1].astype(jnp.int32)
    pad = E_PAD - E
    fill = jnp.full((pad,), DUMMY, jnp.int32)
    src2d = jnp.concatenate([src, fill]).reshape(E_PAD // SUB, SUB)
    dst2d = jnp.concatenate([dst, fill]).reshape(E_PAD // SUB, SUB)
    ei2 = jnp.stack([src2d, dst2d], axis=1)
    z64 = jnp.zeros((ROWS_PER_TILE, HID), VDT)
    z16 = jnp.zeros((ROWS_PER_TILE, DW), jnp.float32)
    ones16 = jnp.ones((SUB, DW), jnp.float32)
    b1 = b1_l.reshape(1, HID)
    b2 = b2_l.reshape(1, HID)

    p1, q1 = _tc_pre(x, W1_l, W1_r, b1)
    part1, degp = _sc_agg_deg(p1, ei2, z64, z16, ones16)
    p2, q2 = _tc_mid(part1, degp, q1, W2_l, W2_r, b2)
    (part2,) = _sc_agg(p2, ei2, z64)
    return _tc_out(part2, degp, q2)


# speedup vs baseline: 16.5784x; 1.0656x over previous
import functools

import jax
import jax.numpy as jnp
from jax import lax
from jax.experimental import pallas as pl
from jax.experimental.pallas import tpu as pltpu
from jax.experimental.pallas import tpu_sc as plsc

N_NODES = 10000
E = 320000
IN_DIM = 128
HID = 64

NC = 2
NS = 16
NW = NC * NS
SUB = 128
K = 8
CHUNK = SUB * K
CHUNKS_PER_W = 10
DW = 8
E_PAD = NW * CHUNK * CHUNKS_PER_W
IDXROWS_PER_W = E_PAD // SUB // NW
N_PAD = 10240
ROWS_PER_TILE = N_PAD // NS
DUMMY = N_NODES
RBLK = 512
VDT = jnp.bfloat16


EBLK = E_PAD // (N_PAD // RBLK)


def _tc_pre_body(x_ref, wl_ref, wr_ref, b_ref, e_ref, p_ref, q_ref, ei_ref):
    xb = x_ref[...]
    p_ref[...] = jnp.dot(
        xb, wl_ref[...], preferred_element_type=jnp.float32).astype(VDT)
    q_ref[...] = (jnp.dot(xb, wr_ref[...], preferred_element_type=jnp.float32)
                  + b_ref[...])
    i = pl.program_id(0)
    col = jax.lax.broadcasted_iota(jnp.int32, (2, EBLK), 1) + i * EBLK
    e = jnp.where(col < E, e_ref[...], DUMMY)
    ei_ref[...] = e.reshape(2, EBLK // SUB, SUB)


def _tc_pre(x, W_l, W_r, b, e):
    grid = (N_PAD // RBLK,)
    return pl.pallas_call(
        _tc_pre_body,
        grid=grid,
        in_specs=[
            pl.BlockSpec((RBLK, IN_DIM), lambda i: (i, 0)),
            pl.BlockSpec((IN_DIM, HID), lambda i: (0, 0)),
            pl.BlockSpec((IN_DIM, HID), lambda i: (0, 0)),
            pl.BlockSpec((1, HID), lambda i: (0, 0)),
            pl.BlockSpec((2, EBLK), lambda i: (0, i)),
        ],
        out_specs=[
            pl.BlockSpec((RBLK, HID), lambda i: (i, 0)),
            pl.BlockSpec((RBLK, HID), lambda i: (i, 0)),
            pl.BlockSpec((2, EBLK // SUB, SUB), lambda i: (0, i, 0)),
        ],
        out_shape=[
            jax.ShapeDtypeStruct((N_PAD, HID), VDT),
            jax.ShapeDtypeStruct((N_PAD, HID), jnp.float32),
            jax.ShapeDtypeStruct((2, E_PAD // SUB, SUB), jnp.int32),
        ],
    )(x, W_l, W_r, b, e)


def _tc_mid_body(part_ref, degp_ref, q1_ref, wl_ref, wr_ref, b_ref,
                 p2_ref, q2_ref):
    agg = (part_ref[0].astype(jnp.float32)
           + part_ref[1].astype(jnp.float32))
    deg = degp_ref[0, :, 0] + degp_ref[1, :, 0]
    mean = agg / jnp.maximum(deg, 1.0)[:, None]
    h = jnp.maximum(mean + q1_ref[...], 0.0)
    p2_ref[...] = jnp.dot(
        h, wl_ref[...], preferred_element_type=jnp.float32).astype(VDT)
    q2_ref[...] = (jnp.dot(h, wr_ref[...], preferred_element_type=jnp.float32)
                   + b_ref[...])


def _tc_mid(part, degp, q1, W_l, W_r, b):
    grid = (N_PAD // RBLK,)
    return pl.pallas_call(
        _tc_mid_body,
        grid=grid,
        in_specs=[
            pl.BlockSpec((NC, RBLK, HID), lambda i: (0, i, 0)),
            pl.BlockSpec((NC, RBLK, DW), lambda i: (0, i, 0)),
            pl.BlockSpec((RBLK, HID), lambda i: (i, 0)),
            pl.BlockSpec((HID, HID), lambda i: (0, 0)),
            pl.BlockSpec((HID, HID), lambda i: (0, 0)),
            pl.BlockSpec((1, HID), lambda i: (0, 0)),
        ],
        out_specs=[
            pl.BlockSpec((RBLK, HID), lambda i: (i, 0)),
            pl.BlockSpec((RBLK, HID), lambda i: (i, 0)),
        ],
        out_shape=[
            jax.ShapeDtypeStruct((N_PAD, HID), VDT),
            jax.ShapeDtypeStruct((N_PAD, HID), jnp.float32),
        ],
    )(part, degp, q1, W_l, W_r, b)


def _tc_out_body(part_ref, degp_ref, q2_ref, z_ref):
    agg = (part_ref[0].astype(jnp.float32)
           + part_ref[1].astype(jnp.float32))
    deg = degp_ref[0, :, 0] + degp_ref[1, :, 0]
    z_ref[...] = agg / jnp.maximum(deg, 1.0)[:, None] + q2_ref[...]


def _tc_out(part, degp, q2):
    oblk = 400
    grid = (N_NODES // oblk,)
    return pl.pallas_call(
        _tc_out_body,
        grid=grid,
        in_specs=[
            pl.BlockSpec((NC, oblk, HID), lambda i: (0, i, 0)),
            pl.BlockSpec((NC, oblk, DW), lambda i: (0, i, 0)),
            pl.BlockSpec((oblk, HID), lambda i: (i, 0)),
        ],
        out_specs=pl.BlockSpec((oblk, HID), lambda i: (i, 0)),
        out_shape=jax.ShapeDtypeStruct((N_NODES, HID), jnp.float32),
    )(part, degp, q2)


def _make_sc_agg(with_deg):
    mesh = plsc.VectorSubcoreMesh(core_axis_name="c", subcore_axis_name="s")
    out_type = [jax.ShapeDtypeStruct((NC, N_PAD, HID), VDT)]
    scratch = [
        pltpu.VMEM((2, K, SUB), jnp.int32),
        pltpu.VMEM((2, K, SUB), jnp.int32),
        pltpu.VMEM((CHUNK, HID), VDT),
        pltpu.VMEM((CHUNK, HID), VDT),
        pltpu.VMEM_SHARED((N_PAD, HID), VDT),
        pltpu.VMEM_SHARED((N_PAD, HID), VDT),
        pltpu.SemaphoreType.DMA,
        pltpu.SemaphoreType.DMA,
        pltpu.SemaphoreType.DMA,
        pltpu.SemaphoreType.DMA,
    ]
    if with_deg:
        out_type.append(jax.ShapeDtypeStruct((NC, N_PAD, DW), jnp.float32))
        scratch += [
            pltpu.VMEM((SUB, DW), jnp.float32),
            pltpu.VMEM_SHARED((N_PAD, DW), jnp.float32),
        ]

    @functools.partial(
        pl.kernel,
        mesh=mesh,
        out_type=out_type,
        scratch_types=scratch,
        compiler_params=pltpu.CompilerParams(use_tc_tiling_on_sc=False),
    )
    def k(*refs):
        if with_deg:
            (table_h, ei_h, z64_h, z16_h, ones_h, part_out, deg_out,
             idx0, idx1, rows0, rows1, acc, table_sp, sg0, sg1, ss0, ss1,
             ones_v, dacc) = refs
        else:
            (table_h, ei_h, z64_h, part_out,
             idx0, idx1, rows0, rows1, acc, table_sp, sg0, sg1, ss0,
             ss1) = refs
        c = lax.axis_index("c")
        s = lax.axis_index("s")
        w = s * NC + c
        tile_rows = pl.ds(s * ROWS_PER_TILE, ROWS_PER_TILE)
        pro = [
            pltpu.async_copy(z64_h, acc.at[tile_rows], sg0),
            pltpu.async_copy(table_h.at[tile_rows], table_sp.at[tile_rows],
                             sg1),
        ]
        if with_deg:
            pro.append(pltpu.async_copy(z16_h, dacc.at[tile_rows], ss0))
            pro.append(pltpu.async_copy(ones_h, ones_v, ss1))
        for d in pro:
            d.wait()
        plsc.subcore_barrier()

        idx = (idx0, idx1)
        rows = (rows0, rows1)
        sg = (sg0, sg1)
        ss = (ss0, ss1)

        def copy_idx(b, t):
            row0 = w * IDXROWS_PER_W + t * K
            pltpu.sync_copy(ei_h.at[:, pl.ds(row0, K)], idx[b])

        def issue_gathers(b):
            return [
                pltpu.async_copy(table_sp.at[idx[b].at[0, j]],
                                 rows[b].at[pl.ds(j * SUB, SUB)], sg[b])
                for j in range(K)
            ]

        def issue_scatters(b):
            ds = []
            for j in range(K):
                ds.append(pltpu.async_copy(rows[b].at[pl.ds(j * SUB, SUB)],
                                           acc.at[idx[b].at[1, j]], ss[b],
                                           add=True))
                if with_deg:
                    ds.append(pltpu.async_copy(ones_v,
                                               dacc.at[idx[b].at[1, j]],
                                               ss[b], add=True))
            return ds

        copy_idx(0, 0)
        dg = [None, None]
        dsc = [None, None]
        dg[0] = issue_gathers(0)
        for t in range(CHUNKS_PER_W):
            b = t % 2
            nb = 1 - b
            if t + 1 < CHUNKS_PER_W:
                if dsc[nb] is not None:
                    for d in dsc[nb]:
                        d.wait()
                copy_idx(nb, t + 1)
                dg[nb] = issue_gathers(nb)
            for d in dg[b]:
                d.wait()
            dsc[b] = issue_scatters(b)
        for bb in (0, 1):
            for d in dsc[bb]:
                d.wait()
        plsc.subcore_barrier()
        epi = [pltpu.async_copy(acc.at[tile_rows],
                                part_out.at[c].at[tile_rows], sg0)]
        if with_deg:
            epi.append(pltpu.async_copy(dacc.at[tile_rows],
                                        deg_out.at[c].at[tile_rows], sg1))
        for d in epi:
            d.wait()

    return k


_sc_agg_deg = _make_sc_agg(True)
_sc_agg = _make_sc_agg(False)


def kernel(x, edge_index, W1_l, b1_l, W1_r, W2_l, b2_l, W2_r):
    e32 = edge_index.astype(jnp.int32)
    z64 = jnp.zeros((ROWS_PER_TILE, HID), VDT)
    z16 = jnp.zeros((ROWS_PER_TILE, DW), jnp.float32)
    ones16 = jnp.ones((SUB, DW), jnp.float32)
    b1 = b1_l.reshape(1, HID)
    b2 = b2_l.reshape(1, HID)

    p1, q1, ei2 = _tc_pre(x, W1_l, W1_r, b1, e32)
    part1, degp = _sc_agg_deg(p1, ei2, z64, z16, ones16)
    p2, q2 = _tc_mid(part1, degp, q1, W2_l, W2_r, b2)
    (part2,) = _sc_agg(p2, ei2, z64)
    return _tc_out(part2, degp, q2)


# speedup vs baseline: 16.7650x; 1.0113x over previous
import functools

import jax
import jax.numpy as jnp
from jax import lax
from jax.experimental import pallas as pl
from jax.experimental.pallas import tpu as pltpu
from jax.experimental.pallas import tpu_sc as plsc

N_NODES = 10000
E = 320000
IN_DIM = 128
HID = 64

NC = 2
NS = 16
NW = NC * NS
SUB = 128
K = 8
CHUNK = SUB * K
CHUNKS_PER_W = 10
DW = 8
E_PAD = NW * CHUNK * CHUNKS_PER_W
IDXROWS_PER_W = E_PAD // SUB // NW
N_PAD = 10240
ROWS_PER_TILE = N_PAD // NS
DUMMY = N_NODES
RBLK = 512
VDT = jnp.bfloat16


EBLK = E_PAD // (N_PAD // RBLK)


def _tc_pre_body(x_ref, wl_ref, wr_ref, b_ref, e_ref, p_ref, q_ref, ei_ref):
    xb = x_ref[...]
    p_ref[...] = jnp.dot(
        xb, wl_ref[...], preferred_element_type=jnp.float32).astype(VDT)
    q_ref[...] = (jnp.dot(xb, wr_ref[...], preferred_element_type=jnp.float32)
                  + b_ref[...])
    i = pl.program_id(0)
    col = jax.lax.broadcasted_iota(jnp.int32, (2, EBLK), 1) + i * EBLK
    e = jnp.where(col < E, e_ref[...], DUMMY)
    ei_ref[...] = e.reshape(2, EBLK // SUB, SUB)


def _tc_pre(x, W_l, W_r, b, e):
    grid = (N_PAD // RBLK,)
    return pl.pallas_call(
        _tc_pre_body,
        grid=grid,
        in_specs=[
            pl.BlockSpec((RBLK, IN_DIM), lambda i: (i, 0)),
            pl.BlockSpec((IN_DIM, HID), lambda i: (0, 0)),
            pl.BlockSpec((IN_DIM, HID), lambda i: (0, 0)),
            pl.BlockSpec((1, HID), lambda i: (0, 0)),
            pl.BlockSpec((2, EBLK), lambda i: (0, i)),
        ],
        out_specs=[
            pl.BlockSpec((RBLK, HID), lambda i: (i, 0)),
            pl.BlockSpec((RBLK, HID), lambda i: (i, 0)),
            pl.BlockSpec((2, EBLK // SUB, SUB), lambda i: (0, i, 0)),
        ],
        out_shape=[
            jax.ShapeDtypeStruct((N_PAD, HID), VDT),
            jax.ShapeDtypeStruct((N_PAD, HID), jnp.float32),
            jax.ShapeDtypeStruct((2, E_PAD // SUB, SUB), jnp.int32),
        ],
    )(x, W_l, W_r, b, e)


def _tc_mid_body(part_ref, degp_ref, q1_ref, wl_ref, wr_ref, b_ref,
                 p2_ref, q2_ref):
    agg = (part_ref[0].astype(jnp.float32)
           + part_ref[1].astype(jnp.float32))
    deg = degp_ref[0, :, 0] + degp_ref[1, :, 0]
    mean = agg / jnp.maximum(deg, 1.0)[:, None]
    h = jnp.maximum(mean + q1_ref[...], 0.0)
    p2_ref[...] = jnp.dot(
        h, wl_ref[...], preferred_element_type=jnp.float32).astype(VDT)
    q2_ref[...] = (jnp.dot(h, wr_ref[...], preferred_element_type=jnp.float32)
                   + b_ref[...])


def _tc_mid(part, degp, q1, W_l, W_r, b):
    grid = (N_PAD // RBLK,)
    return pl.pallas_call(
        _tc_mid_body,
        grid=grid,
        in_specs=[
            pl.BlockSpec((NC, RBLK, HID), lambda i: (0, i, 0)),
            pl.BlockSpec((NC, RBLK, DW), lambda i: (0, i, 0)),
            pl.BlockSpec((RBLK, HID), lambda i: (i, 0)),
            pl.BlockSpec((HID, HID), lambda i: (0, 0)),
            pl.BlockSpec((HID, HID), lambda i: (0, 0)),
            pl.BlockSpec((1, HID), lambda i: (0, 0)),
        ],
        out_specs=[
            pl.BlockSpec((RBLK, HID), lambda i: (i, 0)),
            pl.BlockSpec((RBLK, HID), lambda i: (i, 0)),
        ],
        out_shape=[
            jax.ShapeDtypeStruct((N_PAD, HID), VDT),
            jax.ShapeDtypeStruct((N_PAD, HID), jnp.float32),
        ],
    )(part, degp, q1, W_l, W_r, b)


def _tc_out_body(part_ref, degp_ref, q2_ref, z_ref):
    agg = (part_ref[0].astype(jnp.float32)
           + part_ref[1].astype(jnp.float32))
    deg = degp_ref[0, :, 0] + degp_ref[1, :, 0]
    z_ref[...] = agg / jnp.maximum(deg, 1.0)[:, None] + q2_ref[...]


def _tc_out(part, degp, q2):
    oblk = 400
    grid = (N_NODES // oblk,)
    return pl.pallas_call(
        _tc_out_body,
        grid=grid,
        in_specs=[
            pl.BlockSpec((NC, oblk, HID), lambda i: (0, i, 0)),
            pl.BlockSpec((NC, oblk, DW), lambda i: (0, i, 0)),
            pl.BlockSpec((oblk, HID), lambda i: (i, 0)),
        ],
        out_specs=pl.BlockSpec((oblk, HID), lambda i: (i, 0)),
        out_shape=jax.ShapeDtypeStruct((N_NODES, HID), jnp.float32),
    )(part, degp, q2)


def _make_sc_agg(with_deg):
    mesh = plsc.VectorSubcoreMesh(core_axis_name="c", subcore_axis_name="s")
    out_type = [jax.ShapeDtypeStruct((NC, N_PAD, HID), VDT)]
    scratch = [
        pltpu.VMEM((2, K, SUB), jnp.int32),
        pltpu.VMEM((2, K, SUB), jnp.int32),
        pltpu.VMEM((CHUNK, HID), VDT),
        pltpu.VMEM((CHUNK, HID), VDT),
        pltpu.VMEM_SHARED((N_PAD, HID), VDT),
        pltpu.VMEM_SHARED((N_PAD, HID), VDT),
        pltpu.SemaphoreType.DMA,
        pltpu.SemaphoreType.DMA,
        pltpu.SemaphoreType.DMA,
        pltpu.SemaphoreType.DMA,
    ]
    if with_deg:
        out_type.append(jax.ShapeDtypeStruct((NC, N_PAD, DW), jnp.float32))
        scratch += [
            pltpu.VMEM((SUB, DW), jnp.float32),
            pltpu.VMEM_SHARED((N_PAD, DW), jnp.float32),
        ]

    @functools.partial(
        pl.kernel,
        mesh=mesh,
        out_type=out_type,
        scratch_types=scratch,
        compiler_params=pltpu.CompilerParams(use_tc_tiling_on_sc=False),
    )
    def k(*refs):
        if with_deg:
            (table_h, ei_h, z64_h, z16_h, ones_h, part_out, deg_out,
             idx0, idx1, rows0, rows1, acc, table_sp, sg0, sg1, ss0, ss1,
             ones_v, dacc) = refs
        else:
            (table_h, ei_h, z64_h, part_out,
             idx0, idx1, rows0, rows1, acc, table_sp, sg0, sg1, ss0,
             ss1) = refs
        c = lax.axis_index("c")
        s = lax.axis_index("s")
        w = s * NC + c
        tile_rows = pl.ds(s * ROWS_PER_TILE, ROWS_PER_TILE)
        pro = [
            pltpu.async_copy(z64_h, acc.at[tile_rows], sg0),
            pltpu.async_copy(table_h.at[tile_rows], table_sp.at[tile_rows],
                             sg1),
        ]
        if with_deg:
            pro.append(pltpu.async_copy(z16_h, dacc.at[tile_rows], ss0))
            pro.append(pltpu.async_copy(ones_h, ones_v, ss1))
        for d in pro:
            d.wait()
        plsc.subcore_barrier()

        idx = (idx0, idx1)
        rows = (rows0, rows1)
        sg = (sg0, sg1)
        ss = (ss0, ss1)

        def copy_idx(b, t):
            row0 = w * IDXROWS_PER_W + t * K
            pltpu.sync_copy(ei_h.at[:, pl.ds(row0, K)], idx[b])

        def issue_gathers(b):
            return [
                pltpu.async_copy(table_sp.at[idx[b].at[0, j]],
                                 rows[b].at[pl.ds(j * SUB, SUB)], sg[b])
                for j in range(K)
            ]

        def issue_scatters(b):
            ds = []
            for j in range(K):
                ds.append(pltpu.async_copy(rows[b].at[pl.ds(j * SUB, SUB)],
                                           acc.at[idx[b].at[1, j]], ss[b],
                                           add=True))
                if with_deg:
                    ds.append(pltpu.async_copy(ones_v,
                                               dacc.at[idx[b].at[1, j]],
                                               ss[b], add=True))
            return ds

        def wait_gathers(b):
            for j in range(K):
                pltpu.make_async_copy(
                    table_sp.at[idx[b].at[0, j]],
                    rows[b].at[pl.ds(j * SUB, SUB)], sg[b]).wait()

        def wait_scatters(b):
            for j in range(K):
                pltpu.make_async_copy(
                    rows[b].at[pl.ds(j * SUB, SUB)],
                    acc.at[idx[b].at[1, j]], ss[b]).wait()
                if with_deg:
                    pltpu.make_async_copy(
                        ones_v, dacc.at[idx[b].at[1, j]], ss[b]).wait()

        npair = CHUNKS_PER_W // 2
        copy_idx(0, 0)
        issue_gathers(0)

        def pair_body(i, carry):
            @pl.when(i > 0)
            def _():
                wait_scatters(1)
            copy_idx(1, 2 * i + 1)
            issue_gathers(1)
            wait_gathers(0)
            issue_scatters(0)
            wait_scatters(0)
            @pl.when(i < npair - 1)
            def _():
                copy_idx(0, 2 * i + 2)
                issue_gathers(0)
            wait_gathers(1)
            issue_scatters(1)
            return carry

        lax.fori_loop(0, npair, pair_body, 0)
        wait_scatters(1)
        plsc.subcore_barrier()
        epi = [pltpu.async_copy(acc.at[tile_rows],
                                part_out.at[c].at[tile_rows], sg0)]
        if with_deg:
            epi.append(pltpu.async_copy(dacc.at[tile_rows],
                                        deg_out.at[c].at[tile_rows], sg1))
        for d in epi:
            d.wait()

    return k


_sc_agg_deg = _make_sc_agg(True)
_sc_agg = _make_sc_agg(False)


def kernel(x, edge_index, W1_l, b1_l, W1_r, W2_l, b2_l, W2_r):
    e32 = edge_index.astype(jnp.int32)
    z64 = jnp.zeros((ROWS_PER_TILE, HID), VDT)
    z16 = jnp.zeros((ROWS_PER_TILE, DW), jnp.float32)
    ones16 = jnp.ones((SUB, DW), jnp.float32)
    b1 = b1_l.reshape(1, HID)
    b2 = b2_l.reshape(1, HID)

    p1, q1, ei2 = _tc_pre(x, W1_l, W1_r, b1, e32)
    part1, degp = _sc_agg_deg(p1, ei2, z64, z16, ones16)
    p2, q2 = _tc_mid(part1, degp, q1, W2_l, W2_r, b2)
    (part2,) = _sc_agg(p2, ei2, z64)
    return _tc_out(part2, degp, q2)
